# baseline (speedup 1.0000x reference)
import jax
import jax.numpy as jnp
from jax.experimental import pallas as pl
from jax.experimental.pallas import tpu as pltpu

_NCI = 8
_NCO = 4


def _dot(a, b):
    return jnp.dot(a, b, preferred_element_type=jnp.float32)


def _dott(a, b):
    return jax.lax.dot_general(a, b, (((0,), (0,)), ((), ())),
                               preferred_element_type=jnp.float32)


def _gcn_body(x_hbm, w1_ref, b1_ref, w2_ref, b2_ref, w3_ref, b3_ref,
              wc_ref, bc_ref, out_hbm, h_hbm,
              xn_ref, h3_ref, o3_ref, in_sems, out_sems):
    n, d = xn_ref.shape
    ci = n // _NCI
    co = n // _NCO

    def in_copy(c):
        sl = pl.ds(c * ci, ci)
        return pltpu.make_async_copy(x_hbm.at[sl, :], xn_ref.at[sl, :],
                                     in_sems.at[c])

    for c in range(_NCI):
        in_copy(c).start()

    w1 = w1_ref[...]
    t1 = jnp.zeros((d, w1.shape[1]), jnp.float32)
    for c in range(_NCI):
        in_copy(c).wait()
        sl = pl.ds(c * ci, ci)
        x = xn_ref[sl, :]
        norm = jnp.sqrt(jnp.sum(x * x, axis=1, keepdims=True))
        xn = x / jnp.maximum(norm, 1e-8)
        xn_ref[sl, :] = xn
        t1 = t1 + _dott(xn, _dot(x, w1))

    xn = xn_ref[...]
    h1 = _dot(xn, t1) + b1_ref[...]
    h1 = jnp.where(h1 >= 0, h1, 0.25 * h1)
    t2 = _dott(xn, _dot(h1, w2_ref[...]))
    h2 = _dot(xn, t2) + b2_ref[...]
    h2 = jnp.where(h2 >= 0, h2, 0.25 * h2)
    t3 = _dott(xn, _dot(h2, w3_ref[...]))

    b3 = b3_ref[...]
    wc = wc_ref[...]
    bc = bc_ref[...]
    for c in range(_NCO):
        sl = pl.ds(c * co, co)
        hh = _dot(xn_ref[sl, :], t3) + b3
        hh = jnp.where(hh >= 0, hh, 0.25 * hh)
        h3_ref[sl, :] = hh
        o3_ref[sl, :] = _dot(hh, wc) + bc
        pltpu.make_async_copy(h3_ref.at[sl, :], h_hbm.at[sl, :],
                              out_sems.at[2 * c]).start()
        pltpu.make_async_copy(o3_ref.at[sl, :], out_hbm.at[sl, :],
                              out_sems.at[2 * c + 1]).start()

    for c in range(_NCO):
        sl = pl.ds(c * co, co)
        pltpu.make_async_copy(h3_ref.at[sl, :], h_hbm.at[sl, :],
                              out_sems.at[2 * c]).wait()
        pltpu.make_async_copy(o3_ref.at[sl, :], out_hbm.at[sl, :],
                              out_sems.at[2 * c + 1]).wait()


def kernel(x, W1, b1, W2, b2, W3, b3, Wc, bc):
    n, d = x.shape
    do = Wc.shape[1]
    vspec = pl.BlockSpec(memory_space=pltpu.MemorySpace.VMEM)
    aspec = pl.BlockSpec(memory_space=pltpu.MemorySpace.HBM)

    out, h = pl.pallas_call(
        _gcn_body,
        in_specs=[aspec] + [vspec] * 8,
        out_specs=(aspec, aspec),
        out_shape=(
            jax.ShapeDtypeStruct((n, do), jnp.float32),
            jax.ShapeDtypeStruct((n, do), jnp.float32),
        ),
        scratch_shapes=[
            pltpu.VMEM((n, d), jnp.float32),
            pltpu.VMEM((n, do), jnp.float32),
            pltpu.VMEM((n, do), jnp.float32),
            pltpu.SemaphoreType.DMA((_NCI,)),
            pltpu.SemaphoreType.DMA((2 * _NCO,)),
        ],
    )(x, W1, b1[0, 0][None, :], W2, b2[0, 0][None, :],
      W3, b3[0, 0][None, :], Wc, bc[None, :])
    return (out, h)


# speedup vs baseline: 1.0993x; 1.0993x over previous
import jax
import jax.numpy as jnp
from jax.experimental import pallas as pl
from jax.experimental.pallas import tpu as pltpu

_NCI = 2
_NCO = 2


def _dot(a, b):
    return jnp.dot(a, b, preferred_element_type=jnp.float32)


def _dott(a, b):
    return jax.lax.dot_general(a, b, (((0,), (0,)), ((), ())),
                               preferred_element_type=jnp.float32)


def _gcn_body(x_hbm, w1_ref, b1_ref, w2_ref, b2_ref, w3_ref, b3_ref,
              wc_ref, bc_ref, out_hbm, h_hbm,
              xn_ref, h3_ref, o3_ref, in_sems, out_sems):
    n, d = xn_ref.shape
    ci = n // _NCI
    co = n // _NCO

    def in_copy(c):
        sl = pl.ds(c * ci, ci)
        return pltpu.make_async_copy(x_hbm.at[sl, :], xn_ref.at[sl, :],
                                     in_sems.at[c])

    for c in range(_NCI):
        in_copy(c).start()

    w1 = w1_ref[...]
    t1 = jnp.zeros((d, w1.shape[1]), jnp.float32)
    for c in range(_NCI):
        in_copy(c).wait()
        sl = pl.ds(c * ci, ci)
        x = xn_ref[sl, :]
        norm = jnp.sqrt(jnp.sum(x * x, axis=1, keepdims=True))
        xn = x / jnp.maximum(norm, 1e-8)
        xn_ref[sl, :] = xn
        t1 = t1 + _dott(xn, _dot(x, w1))

    xn = xn_ref[...]
    h1 = _dot(xn, t1) + b1_ref[...]
    h1 = jnp.where(h1 >= 0, h1, 0.25 * h1)
    t2 = _dott(xn, _dot(h1, w2_ref[...]))
    h2 = _dot(xn, t2) + b2_ref[...]
    h2 = jnp.where(h2 >= 0, h2, 0.25 * h2)
    t3 = _dott(xn, _dot(h2, w3_ref[...]))

    b3 = b3_ref[...]
    wc = wc_ref[...]
    bc = bc_ref[...]
    for c in range(_NCO):
        sl = pl.ds(c * co, co)
        hh = _dot(xn_ref[sl, :], t3) + b3
        hh = jnp.where(hh >= 0, hh, 0.25 * hh)
        h3_ref[sl, :] = hh
        o3_ref[sl, :] = _dot(hh, wc) + bc
        pltpu.make_async_copy(h3_ref.at[sl, :], h_hbm.at[sl, :],
                              out_sems.at[2 * c]).start()
        pltpu.make_async_copy(o3_ref.at[sl, :], out_hbm.at[sl, :],
                              out_sems.at[2 * c + 1]).start()

    for c in range(_NCO):
        sl = pl.ds(c * co, co)
        pltpu.make_async_copy(h3_ref.at[sl, :], h_hbm.at[sl, :],
                              out_sems.at[2 * c]).wait()
        pltpu.make_async_copy(o3_ref.at[sl, :], out_hbm.at[sl, :],
                              out_sems.at[2 * c + 1]).wait()


def kernel(x, W1, b1, W2, b2, W3, b3, Wc, bc):
    n, d = x.shape
    do = Wc.shape[1]
    vspec = pl.BlockSpec(memory_space=pltpu.MemorySpace.VMEM)
    aspec = pl.BlockSpec(memory_space=pltpu.MemorySpace.HBM)

    out, h = pl.pallas_call(
        _gcn_body,
        in_specs=[aspec] + [vspec] * 8,
        out_specs=(aspec, aspec),
        out_shape=(
            jax.ShapeDtypeStruct((n, do), jnp.float32),
            jax.ShapeDtypeStruct((n, do), jnp.float32),
        ),
        scratch_shapes=[
            pltpu.VMEM((n, d), jnp.float32),
            pltpu.VMEM((n, do), jnp.float32),
            pltpu.VMEM((n, do), jnp.float32),
            pltpu.SemaphoreType.DMA((_NCI,)),
            pltpu.SemaphoreType.DMA((2 * _NCO,)),
        ],
    )(x, W1, b1[0, 0][None, :], W2, b2[0, 0][None, :],
      W3, b3[0, 0][None, :], Wc, bc[None, :])
    return (out, h)


# speedup vs baseline: 1.2036x; 1.0949x over previous
import jax
import jax.numpy as jnp
from jax.experimental import pallas as pl


def _dot(a, b):
    return jnp.dot(a, b, preferred_element_type=jnp.float32)


def _dott(a, b):
    return jax.lax.dot_general(a, b, (((0,), (0,)), ((), ())),
                               preferred_element_type=jnp.float32)


def _lrelu(v):
    return jnp.maximum(v, 0.25 * v)


def _gcn_body(x_ref, w1_ref, b1_ref, w2_ref, b2_ref, w3_ref, b3_ref,
              wc_ref, bc_ref, out_ref, h_ref):
    x = x_ref[...]
    ss = jnp.sum(x * x, axis=1, keepdims=True)
    xn = x * jax.lax.rsqrt(jnp.maximum(ss, 1e-16))

    t1 = _dot(_dott(xn, x), w1_ref[...])
    h1 = _lrelu(_dot(xn, t1) + b1_ref[...])
    t2 = _dot(_dott(xn, h1), w2_ref[...])
    h2 = _lrelu(_dot(xn, t2) + b2_ref[...])
    t3 = _dot(_dott(xn, h2), w3_ref[...])
    h3 = _lrelu(_dot(xn, t3) + b3_ref[...])

    h_ref[...] = h3
    out_ref[...] = _dot(h3, wc_ref[...]) + bc_ref[...]


def kernel(x, W1, b1, W2, b2, W3, b3, Wc, bc):
    n, _ = x.shape
    do = Wc.shape[1]
    out, h = pl.pallas_call(
        _gcn_body,
        out_shape=(
            jax.ShapeDtypeStruct((n, do), jnp.float32),
            jax.ShapeDtypeStruct((n, do), jnp.float32),
        ),
    )(x, W1, b1[0, 0][None, :], W2, b2[0, 0][None, :],
      W3, b3[0, 0][None, :], Wc, bc[None, :])
    return (out, h)
